# Initial kernel scaffold; baseline (speedup 1.0000x reference)
import jax
import jax.numpy as jnp
from jax.experimental import pallas as pl


def kernel(unary, binary, edge_index, edge_weight, unary_clause_weights, binary_clause_weights):
    raise NotImplementedError("write your pallas kernel here")


# speedup vs baseline: 6.4757x; 6.4757x over previous
import functools

import jax
import jax.numpy as jnp
from jax import lax
from jax.experimental import pallas as pl
from jax.experimental.pallas import tpu as pltpu
from jax.experimental.pallas import tpu_sc as plsc

N_NODES = 10000
N_EDGES = 160000
N_UNARY = 128

NTILES = 16
SPAN = 640
SPAN_LAST = N_NODES - 15 * SPAN
NPAD = NTILES * SPAN
TFLAT = NPAD * 4
P1C = 20
EPT = N_EDGES // NTILES
EC = 400
NCHUNK = EPT // EC
GPC = EC * 4 // 16
NSLICE = 8
SLICE_N = NPAD // NSLICE
PORT_N = SLICE_N // NTILES
PORT_W = PORT_N * 4
SENT = 1e30
MASK_HI = -65536


def _build():
    mesh = plsc.VectorSubcoreMesh(core_axis_name="c", subcore_axis_name="s")

    @functools.partial(
        pl.kernel,
        mesh=mesh,
        compiler_params=pltpu.CompilerParams(needs_layout_passes=False),
        out_type=[
            jax.ShapeDtypeStruct((N_NODES * N_UNARY,), jnp.float32),
            jax.ShapeDtypeStruct((N_EDGES * 4,), jnp.float32),
        ],
        scratch_types=[
            pltpu.VMEM((NPAD * 2,), jnp.int32),
            pltpu.VMEM((TFLAT,), jnp.float32),
            pltpu.VMEM((TFLAT,), jnp.float32),
            pltpu.VMEM((P1C * N_UNARY,), jnp.float32),
            pltpu.VMEM((NSLICE * PORT_W,), jnp.float32),
            pltpu.VMEM((SPAN * 2,), jnp.int32),
            pltpu.VMEM((NTILES * PORT_W,), jnp.float32),
            pltpu.VMEM((EC,), jnp.int32),
            pltpu.VMEM((EC,), jnp.int32),
            pltpu.VMEM((EC,), jnp.float32),
            pltpu.VMEM((EC * 4,), jnp.float32),
            pltpu.VMEM((EC * 4,), jnp.float32),
            pltpu.VMEM((16,), jnp.float32),
            pltpu.VMEM((16,), jnp.float32),
            pltpu.VMEM_SHARED((NTILES * SPAN * 2,), jnp.int32),
            pltpu.VMEM_SHARED((NTILES * 2 * NTILES * PORT_W,), jnp.float32),
            pltpu.SemaphoreType.DMA,
        ],
    )
    def k(unary, idx1, idx2, ew, binary, wlane, wb16, out_u, out_b,
          u4p, t1, t2, rowc, piece, u4piece, comb, i1c, i2c, ewc, binc, boc,
          wl_v, wb_v, u4_sh, sb, sem):
        core = lax.axis_index("c")
        w = lax.axis_index("s")

        @pl.when(core == 0)
        def _body():
            iota = lax.iota(jnp.int32, 16)
            pltpu.sync_copy(wlane, wl_v)
            pltpu.sync_copy(wb16, wb_v)
            wlv = wl_v[...]
            slane = (2 * (iota & 1) - 1).astype(jnp.float32)

            def enhance_row(r):
                v = rowc[pl.ds(r * N_UNARY, 16)]
                vsw = plsc.load_gather(rowc, [r * N_UNARY + (iota ^ 1)])
                arg = slane * (v + vsw)
                sig = 1.0 / (1.0 + jnp.exp(-arg))
                return v, vsw, sig

            nch1 = jnp.where(w == 15, SPAN_LAST // P1C, SPAN // P1C)
            row0 = w * SPAN

            def p1_chunk(kk, _):
                base = kk * P1C
                pltpu.sync_copy(unary.at[pl.ds((row0 + base) * N_UNARY, P1C * N_UNARY)], rowc)

                def p1_row(r, _):
                    v, vsw, sig = enhance_row(r)
                    u16 = v + wlv * sig
                    u16s = vsw - wlv * (1.0 - sig)
                    lo = lax.shift_right_logical(plsc.bitcast(u16, jnp.int32), 16)
                    hi = plsc.bitcast(u16s, jnp.int32) & MASK_HI
                    plsc.store_scatter(u4piece, [(base + r) * 2 + (iota >> 1)],
                                       lo | hi, mask=(iota & 1) == 0)
                    return _

                lax.fori_loop(0, P1C, p1_row, 0)
                return _

            lax.fori_loop(0, nch1, p1_chunk, 0)
            pltpu.sync_copy(u4piece, u4_sh.at[pl.ds(w * SPAN * 2, SPAN * 2)])
            plsc.subcore_barrier()
            pltpu.sync_copy(u4_sh, u4p)

            def tinit(g, _):
                t1[pl.ds(g * 16, 16)] = jnp.full((16,), SENT, jnp.float32)
                t2[pl.ds(g * 16, 16)] = jnp.full((16,), SENT, jnp.float32)
                return _

            lax.fori_loop(0, TFLAT // 16, tinit, 0)

            wbv = wb_v[...]
            e_of = iota >> 2
            c_of = iota & 3
            lowhalf = (c_of & 1) == 0
            pair = c_of >> 1

            def unpack(ref, i, p):
                word = plsc.load_gather(ref, [i * 2 + p])
                bits = jnp.where(lowhalf, lax.shift_left(word, 16), word & MASK_HI)
                return plsc.bitcast(bits, jnp.float32)

            def e_chunk(kk, _):
                base = w * EPT + kk * EC
                cp1 = pltpu.async_copy(idx1.at[pl.ds(base, EC)], i1c, sem)
                cp2 = pltpu.async_copy(idx2.at[pl.ds(base, EC)], i2c, sem)
                cp3 = pltpu.async_copy(ew.at[pl.ds(base, EC)], ewc, sem)
                cp4 = pltpu.async_copy(binary.at[pl.ds(base * 4, EC * 4)], binc, sem)
                cp1.wait(); cp2.wait(); cp3.wait(); cp4.wait()

                def e_group(g, _):
                    e = g * 4 + e_of
                    i1 = plsc.load_gather(i1c, [e])
                    i2 = plsc.load_gather(i2c, [e])
                    wv = plsc.load_gather(ewc, [e])
                    x4 = unpack(u4p, i1, pair)
                    y4 = unpack(u4p, i2, pair)
                    bv = binc[pl.ds(g * 16, 16)]
                    m = jnp.maximum(jnp.maximum(-x4, bv), y4)
                    ea = jnp.exp(-x4 - m)
                    eb = jnp.exp(bv - m)
                    ec = jnp.exp(y4 - m)
                    r = wbv * wv / (ea + eb + ec)
                    boc[pl.ds(g * 16, 16)] = bv + r * eb
                    plsc.store_scatter(t1, [i1 * 4 + c_of], -r * ea)
                    plsc.store_scatter(t2, [i2 * 4 + c_of], r * ec)
                    return _

                lax.fori_loop(0, GPC, e_group, 0)
                pltpu.sync_copy(boc, out_b.at[pl.ds(base * 4, EC * 4)])
                return _

            lax.fori_loop(0, NCHUNK, e_chunk, 0)

            def c_slice(s, _):
                src0 = s * (SLICE_N * 4)
                for o in range(NTILES):
                    pltpu.sync_copy(
                        t1.at[pl.ds(src0 + o * PORT_W, PORT_W)],
                        sb.at[pl.ds(((o * 2 + 0) * NTILES + w) * PORT_W, PORT_W)])
                    pltpu.sync_copy(
                        t2.at[pl.ds(src0 + o * PORT_W, PORT_W)],
                        sb.at[pl.ds(((o * 2 + 1) * NTILES + w) * PORT_W, PORT_W)])
                plsc.subcore_barrier()

                def c_side(side, accum):
                    pltpu.sync_copy(sb.at[pl.ds((w * 2 + side) * NTILES * PORT_W,
                                                NTILES * PORT_W)], comb)

                    def c_group(g, _):
                        acc = jnp.full((16,), SENT, jnp.float32)
                        for t in range(NTILES):
                            v = comb[pl.ds(t * PORT_W + g * 16, 16)]
                            acc = jnp.where(v == SENT, acc, v)
                        d = jnp.where(acc == SENT, 0.0, acc)
                        dst = pl.ds(s * PORT_W + g * 16, 16)
                        piece[dst] = d if not accum else piece[dst] + d
                        return _

                    lax.fori_loop(0, PORT_W // 16, c_group, 0)

                c_side(0, False)
                c_side(1, True)
                plsc.subcore_barrier()
                return _

            lax.fori_loop(0, NSLICE, c_slice, 0)

            def p3_portion(s, _):
                prow = s * SLICE_N + w * PORT_N

                @pl.when(prow < N_NODES)
                def _emit():
                    def p3_chunk(kk, _):
                        base = kk * P1C
                        pltpu.sync_copy(
                            unary.at[pl.ds((prow + base) * N_UNARY, P1C * N_UNARY)], rowc)

                        def p3_row(r, _):
                            v, vsw, sig = enhance_row(r)
                            u16 = v + wlv * sig
                            f4 = plsc.load_gather(
                                piece, [s * PORT_W + (base + r) * 4 + c_of])
                            rowc[pl.ds(r * N_UNARY, 16)] = jnp.where(iota < 4, u16 + f4, u16)
                            return _

                        lax.fori_loop(0, P1C, p3_row, 0)
                        pltpu.sync_copy(
                            rowc, out_u.at[pl.ds((prow + base) * N_UNARY, P1C * N_UNARY)])
                        return _

                    lax.fori_loop(0, PORT_N // P1C, p3_chunk, 0)

                return _

            lax.fori_loop(0, NSLICE, p3_portion, 0)

    return k


def kernel(unary, binary, edge_index, edge_weight, unary_clause_weights, binary_clause_weights):
    idx1 = edge_index[0].astype(jnp.int32)
    idx2 = edge_index[1].astype(jnp.int32)
    wl = jnp.stack([-unary_clause_weights, unary_clause_weights], axis=1).reshape(16)
    wb16 = jnp.tile(binary_clause_weights, 4)
    out_u, out_b = _build()(unary.reshape(-1), idx1, idx2, edge_weight,
                            binary.reshape(-1), wl, wb16)
    return (out_u.reshape(N_NODES, N_UNARY), out_b.reshape(N_EDGES, 4))
